# baseline (speedup 1.0000x reference)
import functools
import jax
import jax.numpy as jnp
from jax.experimental import pallas as pl
from jax.experimental.pallas import tpu as pltpu

_N, _K, _D = 10000, 16, 128
_Q, _NF = 20, 8
_E = _N * _K
_QQ = _Q * _Q
_SCALE = 0.1
_T = 8
_EB = _T * _K
_C0 = 0.5 * _SCALE / (_NF ** 0.5)


def _expand_q(x):
    return jnp.broadcast_to(x[:, :, None], (x.shape[0], _Q, _Q)).reshape(
        x.shape[0], _QQ)


def _tile_q(x):
    return jnp.concatenate([x] * _Q, axis=1)


def _tc_body(nh, eh, ehr, has, mi, mij, wh, bh, wl, bl, wr, br, h_out, j_out):
    h = _SCALE * (jnp.dot(nh[...], wh[...], preferred_element_type=jnp.float32)
                  + bh[...])
    h_out[...] = h * mi[...]

    ehv = eh[...]
    ehrv = ehr[...]
    Lp = jnp.dot(ehv, wl[...], preferred_element_type=jnp.float32) + bl[...]
    Rp = jnp.dot(ehv, wr[...], preferred_element_type=jnp.float32) + br[...]
    Lpr = jnp.dot(ehrv, wl[...], preferred_element_type=jnp.float32) + bl[...]
    Rpr = jnp.dot(ehrv, wr[...], preferred_element_type=jnp.float32) + br[...]
    hasb = has[...]
    acc = jnp.zeros((_EB, _QQ), jnp.float32)
    for f in range(_NF):
        sl = slice(f * _Q, (f + 1) * _Q)
        acc += _expand_q(Lp[:, sl]) * _tile_q(Rp[:, sl])
        acc += hasb * (_expand_q(Rpr[:, sl]) * _tile_q(Lpr[:, sl]))
    j_out[...] = (_C0 * mij[...]) * acc


def _tc_call(nh, ehf, ehr, has, mi, mij, wh, bh, wl, bl, wr, br):
    grid = (_N // _T,)
    return pl.pallas_call(
        _tc_body,
        grid=grid,
        in_specs=[
            pl.BlockSpec((_T, _D), lambda i: (i, 0)),
            pl.BlockSpec((_EB, _D), lambda i: (i, 0)),
            pl.BlockSpec((_EB, _D), lambda i: (i, 0)),
            pl.BlockSpec((_EB, 1), lambda i: (i, 0)),
            pl.BlockSpec((_T, 1), lambda i: (i, 0)),
            pl.BlockSpec((_EB, 1), lambda i: (i, 0)),
            pl.BlockSpec((_D, _Q), lambda i: (0, 0)),
            pl.BlockSpec((1, _Q), lambda i: (0, 0)),
            pl.BlockSpec((_D, _NF * _Q), lambda i: (0, 0)),
            pl.BlockSpec((1, _NF * _Q), lambda i: (0, 0)),
            pl.BlockSpec((_D, _NF * _Q), lambda i: (0, 0)),
            pl.BlockSpec((1, _NF * _Q), lambda i: (0, 0)),
        ],
        out_specs=[
            pl.BlockSpec((_T, _Q), lambda i: (i, 0)),
            pl.BlockSpec((_EB, _QQ), lambda i: (i, 0)),
        ],
        out_shape=[
            jax.ShapeDtypeStruct((_N, _Q), jnp.float32),
            jax.ShapeDtypeStruct((_E, _QQ), jnp.float32),
        ],
    )(nh, ehf, ehr, has, mi, mij, wh, bh, wl, bl, wr, br)


def kernel(node_h, edge_h, edge_idx, mask_i, mask_ij, W_h_w, W_h_b, W_J_w, W_J_b):
    B = node_h.shape[0]
    nh = node_h.reshape(_N, _D)
    ehf = edge_h.reshape(_E, _D)
    ei = edge_idx.reshape(_N, _K)

    q_ids = jnp.arange(_Q)[None, :]
    f_ids = jnp.arange(_NF)[:, None]
    perm_l = (q_ids * 2 * _NF + f_ids).reshape(-1)
    perm_r = (q_ids * 2 * _NF + _NF + f_ids).reshape(-1)
    wl = W_J_w[:, perm_l]
    bl = W_J_b[perm_l].reshape(1, _NF * _Q)
    wr = W_J_w[:, perm_r]
    br = W_J_b[perm_r].reshape(1, _NF * _Q)

    nbrs_of_j = jnp.take(ei, ei.reshape(-1), axis=0).reshape(_N, _K, _K)
    match = nbrs_of_j == jnp.arange(_N, dtype=jnp.int32)[:, None, None]
    has = jnp.any(match, axis=-1)
    k_rev = jnp.argmax(match, axis=-1).astype(jnp.int32)
    r = (ei * _K + jnp.where(has, k_rev, 0)).reshape(-1)
    ehr = jnp.take(ehf, r, axis=0)
    hasf = has.astype(jnp.float32).reshape(_E, 1)

    h, J = _tc_call(
        nh, ehf, ehr, hasf,
        mask_i.reshape(_N, 1), mask_ij.reshape(_E, 1),
        W_h_w, W_h_b.reshape(1, _Q), wl, bl, wr, br,
    )
    return h.reshape(B, _N, _Q), J.reshape(B, _N, _K, _Q, _Q)


# speedup vs baseline: 1.0146x; 1.0146x over previous
import functools
import jax
import jax.numpy as jnp
from jax import lax
from jax.experimental import pallas as pl
from jax.experimental.pallas import tpu as pltpu
from jax.experimental.pallas import tpu_sc as plsc

_N, _K, _D = 10000, 16, 128
_Q, _NF = 20, 8
_E = _N * _K
_QQ = _Q * _Q
_SCALE = 0.1
_T = 8
_EB = _T * _K
_C0 = 0.5 * _SCALE / (_NF ** 0.5)

_NC, _NS = 2, 16
_NW = _NC * _NS
_CHUNK = 128
_E_PAD = 163840
_E_W = _E_PAD // _NW
_NCHUNK = _E_W // _CHUNK


def _sc_body(idx_flat, idx_packed, eh, ehrev_out, has_out,
             jv, gv, nbrs, rv, hv, rows, sem1, sem2):
    wid = lax.axis_index("s") * _NC + lax.axis_index("c")
    base = wid * _E_W
    iota16 = lax.iota(jnp.int32, 16)

    def chunk_body(c, _):
        cbase = base + c * _CHUNK
        pltpu.sync_copy(idx_flat.at[pl.ds(cbase, _CHUNK)], jv)

        def rowid_body(g, _):
            e_loc = g * 16 + iota16
            j16 = plsc.load_gather(jv, [e_loc])
            plsc.store_scatter(gv, [e_loc], lax.shift_right_logical(j16, 3))
            return 0

        lax.fori_loop(0, _CHUNK // 16, rowid_body, 0)
        pltpu.async_copy(idx_packed.at[gv], nbrs, sem1).wait()

        def group_body(g, _):
            e_loc = g * 16 + iota16
            j16 = plsc.load_gather(jv, [e_loc])
            off = (j16 & 7) * _K
            i16 = lax.shift_right_logical(cbase + e_loc, 4)
            kmin = jnp.full((16,), 16, jnp.int32)
            for s in range(_K):
                col = plsc.load_gather(nbrs, [e_loc, off + s])
                kmin = jnp.minimum(kmin, jnp.where(col == i16, s, 16))
            hasv = kmin < 16
            plsc.store_scatter(rv, [e_loc],
                               j16 * _K + jnp.where(hasv, kmin, 0))
            plsc.store_scatter(hv, [e_loc],
                               jnp.where(hasv, 1.0, 0.0).astype(jnp.float32))
            return 0

        lax.fori_loop(0, _CHUNK // 16, group_body, 0)
        pltpu.async_copy(eh.at[rv], rows, sem2).wait()
        pltpu.sync_copy(rows, ehrev_out.at[pl.ds(cbase, _CHUNK)])
        pltpu.sync_copy(hv, has_out.at[pl.ds(cbase, _CHUNK)])
        return 0

    lax.fori_loop(0, _NCHUNK, chunk_body, 0)


def _sc_rev_gather(idx_flat, idx_packed, eh):
    mesh = plsc.VectorSubcoreMesh(core_axis_name="c", subcore_axis_name="s")
    fn = functools.partial(
        pl.kernel, mesh=mesh,
        compiler_params=pltpu.CompilerParams(needs_layout_passes=False),
        out_type=[
            jax.ShapeDtypeStruct((_E_PAD, _D), jnp.float32),
            jax.ShapeDtypeStruct((_E_PAD,), jnp.float32),
        ],
        scratch_types=[
            pltpu.VMEM((_CHUNK,), jnp.int32),
            pltpu.VMEM((_CHUNK,), jnp.int32),
            pltpu.VMEM((_CHUNK, 128), jnp.int32),
            pltpu.VMEM((_CHUNK,), jnp.int32),
            pltpu.VMEM((_CHUNK,), jnp.float32),
            pltpu.VMEM((_CHUNK, _D), jnp.float32),
            pltpu.SemaphoreType.DMA,
            pltpu.SemaphoreType.DMA,
        ],
    )(_sc_body)
    return fn(idx_flat, idx_packed, eh)


def _expand_q(x):
    return jnp.repeat(x, _Q, axis=1)


def _tile_q(x):
    return pltpu.repeat(x, _Q, axis=1)


def _tc_body(nh, eh, ehr, has, mi, mij, wh, bh, wl, bl, wr, br, h_out, j_out):
    h = _SCALE * (jnp.dot(nh[...], wh[...], preferred_element_type=jnp.float32)
                  + bh[...])
    h_out[...] = h * mi[...]

    ehv = eh[...]
    ehrv = ehr[...]
    Lp = jnp.dot(ehv, wl[...], preferred_element_type=jnp.float32) + bl[...]
    Rp = jnp.dot(ehv, wr[...], preferred_element_type=jnp.float32) + br[...]
    Lpr = jnp.dot(ehrv, wl[...], preferred_element_type=jnp.float32) + bl[...]
    Rpr = jnp.dot(ehrv, wr[...], preferred_element_type=jnp.float32) + br[...]
    hasb = has[...]
    acc = jnp.zeros((_EB, _QQ), jnp.float32)
    for f in range(_NF):
        sl = slice(f * _Q, (f + 1) * _Q)
        acc += _expand_q(Lp[:, sl]) * _tile_q(Rp[:, sl])
        acc += hasb * (_expand_q(Rpr[:, sl]) * _tile_q(Lpr[:, sl]))
    j_out[...] = (_C0 * mij[...]) * acc


def _tc_call(nh, ehf, ehr, has, mi, mij, wh, bh, wl, bl, wr, br):
    grid = (_N // _T,)
    return pl.pallas_call(
        _tc_body,
        grid=grid,
        in_specs=[
            pl.BlockSpec((_T, _D), lambda i: (i, 0)),
            pl.BlockSpec((_EB, _D), lambda i: (i, 0)),
            pl.BlockSpec((_EB, _D), lambda i: (i, 0)),
            pl.BlockSpec((_EB, 1), lambda i: (i, 0)),
            pl.BlockSpec((_T, 1), lambda i: (i, 0)),
            pl.BlockSpec((_EB, 1), lambda i: (i, 0)),
            pl.BlockSpec((_D, _Q), lambda i: (0, 0)),
            pl.BlockSpec((1, _Q), lambda i: (0, 0)),
            pl.BlockSpec((_D, _NF * _Q), lambda i: (0, 0)),
            pl.BlockSpec((1, _NF * _Q), lambda i: (0, 0)),
            pl.BlockSpec((_D, _NF * _Q), lambda i: (0, 0)),
            pl.BlockSpec((1, _NF * _Q), lambda i: (0, 0)),
        ],
        out_specs=[
            pl.BlockSpec((_T, _Q), lambda i: (i, 0)),
            pl.BlockSpec((_EB, _QQ), lambda i: (i, 0)),
        ],
        out_shape=[
            jax.ShapeDtypeStruct((_N, _Q), jnp.float32),
            jax.ShapeDtypeStruct((_E, _QQ), jnp.float32),
        ],
    )(nh, ehf, ehr, has, mi, mij, wh, bh, wl, bl, wr, br)


def kernel(node_h, edge_h, edge_idx, mask_i, mask_ij, W_h_w, W_h_b, W_J_w, W_J_b):
    B = node_h.shape[0]
    nh = node_h.reshape(_N, _D)
    ehf = edge_h.reshape(_E, _D)
    ei = edge_idx.reshape(_N, _K)

    q_ids = jnp.arange(_Q)[None, :]
    f_ids = jnp.arange(_NF)[:, None]
    perm_l = (q_ids * 2 * _NF + f_ids).reshape(-1)
    perm_r = (q_ids * 2 * _NF + _NF + f_ids).reshape(-1)
    wl = W_J_w[:, perm_l]
    bl = W_J_b[perm_l].reshape(1, _NF * _Q)
    wr = W_J_w[:, perm_r]
    br = W_J_b[perm_r].reshape(1, _NF * _Q)

    idx_flat = jnp.concatenate(
        [ei.reshape(-1), jnp.zeros((_E_PAD - _E,), jnp.int32)])
    ehr, hasp = _sc_rev_gather(idx_flat, ei.reshape(_E // 128, 128), ehf)
    hasf = hasp.reshape(_E_PAD, 1)

    h, J = _tc_call(
        nh, ehf, ehr, hasf,
        mask_i.reshape(_N, 1), mask_ij.reshape(_E, 1),
        W_h_w, W_h_b.reshape(1, _Q), wl, bl, wr, br,
    )
    return h.reshape(B, _N, _Q), J.reshape(B, _N, _K, _Q, _Q)


# speedup vs baseline: 1.4551x; 1.4341x over previous
import functools
import jax
import jax.numpy as jnp
from jax import lax
from jax.experimental import pallas as pl
from jax.experimental.pallas import tpu as pltpu
from jax.experimental.pallas import tpu_sc as plsc

_N, _K, _D = 10000, 16, 128
_Q, _NF = 20, 8
_E = _N * _K
_QQ = _Q * _Q
_SCALE = 0.1
_T = 16
_EB = _T * _K
_C0 = 0.5 * _SCALE / (_NF ** 0.5)

_NC, _NS = 2, 16
_NW = _NC * _NS
_CHUNK = 128
_E_PAD = 163840
_E_W = _E_PAD // _NW
_NCHUNK = _E_W // _CHUNK


def _sc_body(idx_flat, idx_packed, eh, ehrev_out, has_out,
             jv, gv, nbrs, rv, hv, rows, sem1, sem2):
    wid = lax.axis_index("s") * _NC + lax.axis_index("c")
    base = wid * _E_W
    iota16 = lax.iota(jnp.int32, 16)

    def chunk_body(c, _):
        cbase = base + c * _CHUNK
        pltpu.sync_copy(idx_flat.at[pl.ds(cbase, _CHUNK)], jv)

        def rowid_body(g, _):
            e_loc = g * 16 + iota16
            j16 = plsc.load_gather(jv, [e_loc])
            plsc.store_scatter(gv, [e_loc], lax.shift_right_logical(j16, 3))
            return 0

        lax.fori_loop(0, _CHUNK // 16, rowid_body, 0)
        pltpu.async_copy(idx_packed.at[gv], nbrs, sem1).wait()

        def group_body(g, _):
            e_loc = g * 16 + iota16
            j16 = plsc.load_gather(jv, [e_loc])
            off = (j16 & 7) * _K
            i16 = lax.shift_right_logical(cbase + e_loc, 4)
            kmin = jnp.full((16,), 16, jnp.int32)
            for s in range(_K):
                col = plsc.load_gather(nbrs, [e_loc, off + s])
                kmin = jnp.minimum(kmin, jnp.where(col == i16, s, 16))
            hasv = kmin < 16
            plsc.store_scatter(rv, [e_loc],
                               j16 * _K + jnp.where(hasv, kmin, 0))
            plsc.store_scatter(hv, [e_loc],
                               jnp.where(hasv, 1.0, 0.0).astype(jnp.float32))
            return 0

        lax.fori_loop(0, _CHUNK // 16, group_body, 0)
        pltpu.async_copy(eh.at[rv], rows, sem2).wait()
        pltpu.sync_copy(rows, ehrev_out.at[pl.ds(cbase, _CHUNK)])
        pltpu.sync_copy(hv, has_out.at[pl.ds(cbase, _CHUNK)])
        return 0

    lax.fori_loop(0, _NCHUNK, chunk_body, 0)


def _sc_rev_gather(idx_flat, idx_packed, eh):
    mesh = plsc.VectorSubcoreMesh(core_axis_name="c", subcore_axis_name="s")
    fn = functools.partial(
        pl.kernel, mesh=mesh,
        compiler_params=pltpu.CompilerParams(needs_layout_passes=False),
        out_type=[
            jax.ShapeDtypeStruct((_E_PAD, _D), jnp.float32),
            jax.ShapeDtypeStruct((_E_PAD,), jnp.float32),
        ],
        scratch_types=[
            pltpu.VMEM((_CHUNK,), jnp.int32),
            pltpu.VMEM((_CHUNK,), jnp.int32),
            pltpu.VMEM((_CHUNK, 128), jnp.int32),
            pltpu.VMEM((_CHUNK,), jnp.int32),
            pltpu.VMEM((_CHUNK,), jnp.float32),
            pltpu.VMEM((_CHUNK, _D), jnp.float32),
            pltpu.SemaphoreType.DMA,
            pltpu.SemaphoreType.DMA,
        ],
    )(_sc_body)
    return fn(idx_flat, idx_packed, eh)


def _expand_q(x):
    idx = jnp.broadcast_to(
        (jax.lax.iota(jnp.int32, _QQ) // _Q)[None, :], (x.shape[0], _QQ))
    return jnp.take_along_axis(x, idx, axis=1)


def _tile_q(x):
    return jnp.concatenate([x] * _Q, axis=1)


def _tc_body(nh, eh, ehr, has, mi, mij, wh, bh, wl, bl, wr, br, h_out, j_out):
    h = _SCALE * (jnp.dot(nh[...], wh[...], preferred_element_type=jnp.float32)
                  + bh[...])
    h_out[...] = h * mi[...]

    ehv = eh[...]
    ehrv = ehr[...]
    Lp = jnp.dot(ehv, wl[...], preferred_element_type=jnp.float32) + bl[...]
    Rp = jnp.dot(ehv, wr[...], preferred_element_type=jnp.float32) + br[...]
    Lpr = jnp.dot(ehrv, wl[...], preferred_element_type=jnp.float32) + bl[...]
    Rpr = jnp.dot(ehrv, wr[...], preferred_element_type=jnp.float32) + br[...]
    hasb = has[...]
    acc = jnp.zeros((_EB, _QQ), jnp.float32)
    for f in range(_NF):
        sl = slice(f * _Q, (f + 1) * _Q)
        acc += _expand_q(Lp[:, sl]) * _tile_q(Rp[:, sl])
        acc += hasb * (_expand_q(Rpr[:, sl]) * _tile_q(Lpr[:, sl]))
    j_out[...] = (_C0 * mij[...]) * acc


def _tc_call(nh, ehf, ehr, has, mi, mij, wh, bh, wl, bl, wr, br):
    grid = (_N // _T,)
    return pl.pallas_call(
        _tc_body,
        grid=grid,
        in_specs=[
            pl.BlockSpec((_T, _D), lambda i: (i, 0)),
            pl.BlockSpec((_EB, _D), lambda i: (i, 0)),
            pl.BlockSpec((_EB, _D), lambda i: (i, 0)),
            pl.BlockSpec((_EB, 1), lambda i: (i, 0)),
            pl.BlockSpec((_T, 1), lambda i: (i, 0)),
            pl.BlockSpec((_EB, 1), lambda i: (i, 0)),
            pl.BlockSpec((_D, _Q), lambda i: (0, 0)),
            pl.BlockSpec((1, _Q), lambda i: (0, 0)),
            pl.BlockSpec((_D, _NF * _Q), lambda i: (0, 0)),
            pl.BlockSpec((1, _NF * _Q), lambda i: (0, 0)),
            pl.BlockSpec((_D, _NF * _Q), lambda i: (0, 0)),
            pl.BlockSpec((1, _NF * _Q), lambda i: (0, 0)),
        ],
        out_specs=[
            pl.BlockSpec((_T, _Q), lambda i: (i, 0)),
            pl.BlockSpec((_EB, _QQ), lambda i: (i, 0)),
        ],
        out_shape=[
            jax.ShapeDtypeStruct((_N, _Q), jnp.float32),
            jax.ShapeDtypeStruct((_E, _QQ), jnp.float32),
        ],
    )(nh, ehf, ehr, has, mi, mij, wh, bh, wl, bl, wr, br)


def kernel(node_h, edge_h, edge_idx, mask_i, mask_ij, W_h_w, W_h_b, W_J_w, W_J_b):
    B = node_h.shape[0]
    nh = node_h.reshape(_N, _D)
    ehf = edge_h.reshape(_E, _D)
    ei = edge_idx.reshape(_N, _K)

    q_ids = jnp.arange(_Q)[None, :]
    f_ids = jnp.arange(_NF)[:, None]
    perm_l = (q_ids * 2 * _NF + f_ids).reshape(-1)
    perm_r = (q_ids * 2 * _NF + _NF + f_ids).reshape(-1)
    wl = W_J_w[:, perm_l]
    bl = W_J_b[perm_l].reshape(1, _NF * _Q)
    wr = W_J_w[:, perm_r]
    br = W_J_b[perm_r].reshape(1, _NF * _Q)

    idx_flat = jnp.concatenate(
        [ei.reshape(-1), jnp.zeros((_E_PAD - _E,), jnp.int32)])
    ehr, hasp = _sc_rev_gather(idx_flat, ei.reshape(_E // 128, 128), ehf)
    hasf = hasp.reshape(_E_PAD, 1)

    h, J = _tc_call(
        nh, ehf, ehr, hasf,
        mask_i.reshape(_N, 1), mask_ij.reshape(_E, 1),
        W_h_w, W_h_b.reshape(1, _Q), wl, bl, wr, br,
    )
    return h.reshape(B, _N, _Q), J.reshape(B, _N, _K, _Q, _Q)


# speedup vs baseline: 1.4624x; 1.0050x over previous
import functools
import jax
import jax.numpy as jnp
from jax import lax
from jax.experimental import pallas as pl
from jax.experimental.pallas import tpu as pltpu
from jax.experimental.pallas import tpu_sc as plsc

_N, _K, _D = 10000, 16, 128
_Q, _NF = 20, 8
_E = _N * _K
_QQ = _Q * _Q
_SCALE = 0.1
_T = 16
_EB = _T * _K
_C0 = 0.5 * _SCALE / (_NF ** 0.5)

_NC, _NS = 2, 16
_NW = _NC * _NS
_CHUNK = 128
_E_PAD = 163840
_E_W = _E_PAD // _NW
_NCHUNK = _E_W // _CHUNK


def _sc_body(idx_flat, idx_packed, eh, ehrev_out, has_out,
             jv, gv, nbrs, rv, hv, rows,
             semn0, semn1, seme0, seme1):
    wid = lax.axis_index("s") * _NC + lax.axis_index("c")
    base = wid * _E_W
    iota16 = lax.iota(jnp.int32, 16)
    semn = (semn0, semn1)
    seme = (seme0, seme1)

    def chunk_pair(cc, _):
        cb = (base + cc * 2 * _CHUNK, base + (cc * 2 + 1) * _CHUNK)

        hn = []
        for b in range(2):
            pltpu.sync_copy(idx_flat.at[pl.ds(cb[b], _CHUNK)], jv.at[b])

            def rowid_body(g, _, b=b):
                e_loc = g * 16 + iota16
                j16 = plsc.load_gather(jv.at[b], [e_loc])
                plsc.store_scatter(gv.at[b], [e_loc],
                                   lax.shift_right_logical(j16, 3))
                return 0

            lax.fori_loop(0, _CHUNK // 16, rowid_body, 0)
            hn.append(pltpu.async_copy(idx_packed.at[gv.at[b]],
                                       nbrs.at[b], semn[b]))

        he = []
        for b in range(2):
            hn[b].wait()

            def group_body(g, _, b=b):
                e_loc = g * 16 + iota16
                j16 = plsc.load_gather(jv.at[b], [e_loc])
                off = (j16 & 7) * _K
                i16 = lax.shift_right_logical(cb[b] + e_loc, 4)
                kmin = jnp.full((16,), 16, jnp.int32)
                for s in range(_K):
                    col = plsc.load_gather(nbrs.at[b], [e_loc, off + s])
                    kmin = jnp.minimum(kmin, jnp.where(col == i16, s, 16))
                hasv = kmin < 16
                plsc.store_scatter(rv.at[b], [e_loc],
                                   j16 * _K + jnp.where(hasv, kmin, 0))
                plsc.store_scatter(hv.at[b], [e_loc],
                                   jnp.where(hasv, 1.0, 0.0).astype(jnp.float32))
                return 0

            lax.fori_loop(0, _CHUNK // 16, group_body, 0)
            he.append(pltpu.async_copy(eh.at[rv.at[b]], rows.at[b], seme[b]))

        for b in range(2):
            he[b].wait()
            pltpu.sync_copy(rows.at[b], ehrev_out.at[pl.ds(cb[b], _CHUNK)])
            pltpu.sync_copy(hv.at[b], has_out.at[pl.ds(cb[b], _CHUNK)])
        return 0

    lax.fori_loop(0, _NCHUNK // 2, chunk_pair, 0)


def _sc_rev_gather(idx_flat, idx_packed, eh):
    mesh = plsc.VectorSubcoreMesh(core_axis_name="c", subcore_axis_name="s")
    fn = functools.partial(
        pl.kernel, mesh=mesh,
        compiler_params=pltpu.CompilerParams(needs_layout_passes=False),
        out_type=[
            jax.ShapeDtypeStruct((_E_PAD, _D), jnp.float32),
            jax.ShapeDtypeStruct((_E_PAD,), jnp.float32),
        ],
        scratch_types=[
            pltpu.VMEM((2, _CHUNK), jnp.int32),
            pltpu.VMEM((2, _CHUNK), jnp.int32),
            pltpu.VMEM((2, _CHUNK, 128), jnp.int32),
            pltpu.VMEM((2, _CHUNK), jnp.int32),
            pltpu.VMEM((2, _CHUNK), jnp.float32),
            pltpu.VMEM((2, _CHUNK, _D), jnp.float32),
            pltpu.SemaphoreType.DMA,
            pltpu.SemaphoreType.DMA,
            pltpu.SemaphoreType.DMA,
            pltpu.SemaphoreType.DMA,
        ],
    )(_sc_body)
    return fn(idx_flat, idx_packed, eh)


def _expand_q(x):
    idx = jnp.broadcast_to(
        (jax.lax.iota(jnp.int32, _QQ) // _Q)[None, :], (x.shape[0], _QQ))
    return jnp.take_along_axis(x, idx, axis=1)


def _tile_q(x):
    return jnp.concatenate([x] * _Q, axis=1)


def _tc_body(nh, eh, ehr, has, mi, mij, wh, bh, wl, bl, wr, br, h_out, j_out):
    h = _SCALE * (jnp.dot(nh[...], wh[...], preferred_element_type=jnp.float32)
                  + bh[...])
    h_out[...] = h * mi[...]

    ehv = eh[...]
    ehrv = ehr[...]
    Lp = jnp.dot(ehv, wl[...], preferred_element_type=jnp.float32) + bl[...]
    Rp = jnp.dot(ehv, wr[...], preferred_element_type=jnp.float32) + br[...]
    Lpr = jnp.dot(ehrv, wl[...], preferred_element_type=jnp.float32) + bl[...]
    Rpr = jnp.dot(ehrv, wr[...], preferred_element_type=jnp.float32) + br[...]
    hasb = has[...]
    acc = jnp.zeros((_EB, _QQ), jnp.float32)
    for f in range(_NF):
        sl = slice(f * _Q, (f + 1) * _Q)
        acc += _expand_q(Lp[:, sl]) * _tile_q(Rp[:, sl])
        acc += hasb * (_expand_q(Rpr[:, sl]) * _tile_q(Lpr[:, sl]))
    j_out[...] = (_C0 * mij[...]) * acc


def _tc_call(nh, ehf, ehr, has, mi, mij, wh, bh, wl, bl, wr, br):
    grid = (_N // _T,)
    return pl.pallas_call(
        _tc_body,
        grid=grid,
        in_specs=[
            pl.BlockSpec((_T, _D), lambda i: (i, 0)),
            pl.BlockSpec((_EB, _D), lambda i: (i, 0)),
            pl.BlockSpec((_EB, _D), lambda i: (i, 0)),
            pl.BlockSpec((_EB, 1), lambda i: (i, 0)),
            pl.BlockSpec((_T, 1), lambda i: (i, 0)),
            pl.BlockSpec((_EB, 1), lambda i: (i, 0)),
            pl.BlockSpec((_D, _Q), lambda i: (0, 0)),
            pl.BlockSpec((1, _Q), lambda i: (0, 0)),
            pl.BlockSpec((_D, _NF * _Q), lambda i: (0, 0)),
            pl.BlockSpec((1, _NF * _Q), lambda i: (0, 0)),
            pl.BlockSpec((_D, _NF * _Q), lambda i: (0, 0)),
            pl.BlockSpec((1, _NF * _Q), lambda i: (0, 0)),
        ],
        out_specs=[
            pl.BlockSpec((_T, _Q), lambda i: (i, 0)),
            pl.BlockSpec((_EB, _QQ), lambda i: (i, 0)),
        ],
        out_shape=[
            jax.ShapeDtypeStruct((_N, _Q), jnp.float32),
            jax.ShapeDtypeStruct((_E, _QQ), jnp.float32),
        ],
    )(nh, ehf, ehr, has, mi, mij, wh, bh, wl, bl, wr, br)


def kernel(node_h, edge_h, edge_idx, mask_i, mask_ij, W_h_w, W_h_b, W_J_w, W_J_b):
    B = node_h.shape[0]
    nh = node_h.reshape(_N, _D)
    ehf = edge_h.reshape(_E, _D)
    ei = edge_idx.reshape(_N, _K)

    q_ids = jnp.arange(_Q)[None, :]
    f_ids = jnp.arange(_NF)[:, None]
    perm_l = (q_ids * 2 * _NF + f_ids).reshape(-1)
    perm_r = (q_ids * 2 * _NF + _NF + f_ids).reshape(-1)
    wl = W_J_w[:, perm_l]
    bl = W_J_b[perm_l].reshape(1, _NF * _Q)
    wr = W_J_w[:, perm_r]
    br = W_J_b[perm_r].reshape(1, _NF * _Q)

    idx_flat = jnp.concatenate(
        [ei.reshape(-1), jnp.zeros((_E_PAD - _E,), jnp.int32)])
    ehr, hasp = _sc_rev_gather(idx_flat, ei.reshape(_E // 128, 128), ehf)
    hasf = hasp.reshape(_E_PAD, 1)

    h, J = _tc_call(
        nh, ehf, ehr, hasf,
        mask_i.reshape(_N, 1), mask_ij.reshape(_E, 1),
        W_h_w, W_h_b.reshape(1, _Q), wl, bl, wr, br,
    )
    return h.reshape(B, _N, _Q), J.reshape(B, _N, _K, _Q, _Q)
